# baseline (speedup 1.0000x reference)
import functools

import jax
import jax.numpy as jnp
from jax import lax
from jax.experimental import pallas as pl
from jax.experimental.pallas import tpu as pltpu
from jax.experimental.pallas import tpu_sc as plsc

_N, _E, _D = 10000, 160000, 128
_T = 128
_NT = _E // _T
_NC = 2
_NS = 16
_NW = _NC * _NS
_L = 16


def _h_body(x_ref, w_ref, b_ref, o_ref):
    h = lax.dot_general(x_ref[...], w_ref[...], (((1,), (1,)), ((), ())),
                        preferred_element_type=jnp.float32)
    o_ref[...] = (h + b_ref[...]) * (1.0 / 512.0)


def _edge_body(g_hbm, src_hbm, dst_hbm, u_hbm, v_hbm, out_hbm,
               sidx, didx, gs, gd, vbuf, ubuf, sem):
    wid = lax.axis_index("s") * _NC + lax.axis_index("c")
    ntiles = (_NT - wid + _NW - 1) // _NW

    def tile(i, carry):
        t = wid + i * _NW
        off = t * _T
        pltpu.sync_copy(src_hbm.at[pl.ds(off, _T)], sidx)
        pltpu.sync_copy(dst_hbm.at[pl.ds(off, _T)], didx)
        cs = pltpu.async_copy(g_hbm.at[sidx], gs, sem)
        cd = pltpu.async_copy(g_hbm.at[didx], gd, sem)
        pltpu.sync_copy(v_hbm.at[pl.ds(off, _T)], vbuf)
        pltpu.sync_copy(u_hbm.at[pl.ds(off * 3, _T * 3)],
                        ubuf.at[pl.ds(0, _T * 3)])
        cs.wait()
        cd.wait()

        def edge(e, c2):
            uvec = ubuf[pl.ds(e * 3, _L)]
            uv = [uvec[k] for k in range(3)]
            for c in range(_D // _L):
                s = pl.ds(c * _L, _L)
                dv = gs[e, s] - gd[e, s]
                for k in range(3):
                    vbuf[e, k, s] = vbuf[e, k, s] * 0.5 + uv[k] * dv
            return c2

        lax.fori_loop(0, _T, edge, 0)
        pltpu.sync_copy(vbuf, out_hbm.at[pl.ds(off, _T)])
        return carry

    lax.fori_loop(0, ntiles, tile, 0)


def kernel(v, x, edge_index, u, W, b):
    g = pl.pallas_call(
        _h_body,
        out_shape=jax.ShapeDtypeStruct((_N, _D), jnp.float32),
    )(x, W, b.reshape(1, _D))

    mesh = plsc.VectorSubcoreMesh(core_axis_name="c", subcore_axis_name="s")
    edge_fn = functools.partial(
        pl.kernel,
        mesh=mesh,
        out_type=jax.ShapeDtypeStruct((_E, 3, _D), jnp.float32),
        scratch_types=[
            pltpu.VMEM((_T,), jnp.int32),
            pltpu.VMEM((_T,), jnp.int32),
            pltpu.VMEM((_T, _D), jnp.float32),
            pltpu.VMEM((_T, _D), jnp.float32),
            pltpu.VMEM((_T, 3, _D), jnp.float32),
            pltpu.VMEM((_T * 3 + _L,), jnp.float32),
            pltpu.SemaphoreType.DMA,
        ],
    )(_edge_body)
    return edge_fn(g, edge_index[0], edge_index[1], u.reshape(_E * 3), v)


# speedup vs baseline: 1.1602x; 1.1602x over previous
import functools

import jax
import jax.numpy as jnp
from jax import lax
from jax.experimental import pallas as pl
from jax.experimental.pallas import tpu as pltpu
from jax.experimental.pallas import tpu_sc as plsc

_N, _E, _D = 10000, 160000, 128
_NC = 2
_NS = 16
_NW = _NC * _NS
_EW = _E // _NW
_T = 40
_NT = _EW // _T
_L = 16


def _h_body(x_ref, w_ref, b_ref, o_ref):
    h = lax.dot_general(x_ref[...], w_ref[...], (((1,), (1,)), ((), ())),
                        preferred_element_type=jnp.float32)
    o_ref[...] = (h + b_ref[...]) * (1.0 / 512.0)


def _edge_body(g_hbm, src_hbm, dst_hbm, u_hbm, v_hbm, out_hbm,
               sidx, didx, ubuf, gs, gd, vbuf,
               semu, semin0, semin1, semo0, semo1):
    wid = lax.axis_index("s") * _NC + lax.axis_index("c")
    ebase = wid * _EW
    semin = (semin0, semin1)
    semo = (semo0, semo1)

    pltpu.sync_copy(src_hbm.at[pl.ds(ebase, _EW)], sidx)
    pltpu.sync_copy(dst_hbm.at[pl.ds(ebase, _EW)], didx)
    pltpu.async_copy(u_hbm.at[pl.ds(ebase * 3, _EW * 3)],
                     ubuf.at[pl.ds(0, _EW * 3)], semu).wait()

    def fire_inputs(i, b):
        pltpu.async_copy(g_hbm.at[sidx.at[pl.ds(i * _T, _T)]], gs.at[b],
                         semin[b])
        pltpu.async_copy(g_hbm.at[didx.at[pl.ds(i * _T, _T)]], gd.at[b],
                         semin[b])
        pltpu.async_copy(v_hbm.at[pl.ds(ebase + i * _T, _T)], vbuf.at[b],
                         semin[b])

    def wait_inputs(i, b):
        pltpu.make_async_copy(g_hbm.at[sidx.at[pl.ds(i * _T, _T)]], gs.at[b],
                              semin[b]).wait()
        pltpu.make_async_copy(g_hbm.at[didx.at[pl.ds(i * _T, _T)]], gd.at[b],
                              semin[b]).wait()
        pltpu.make_async_copy(v_hbm.at[pl.ds(ebase + i * _T, _T)], vbuf.at[b],
                              semin[b]).wait()

    def fire_out(i, b):
        pltpu.async_copy(vbuf.at[b], out_hbm.at[pl.ds(ebase + i * _T, _T)],
                         semo[b])

    def wait_out(i, b):
        pltpu.make_async_copy(vbuf.at[b], out_hbm.at[pl.ds(ebase + i * _T, _T)],
                              semo[b]).wait()

    def compute(i, b):
        def edge(e, c2):
            uvec = ubuf[pl.ds((i * _T + e) * 3, _L)]
            uv = [uvec[k] for k in range(3)]
            for c in range(_D // _L):
                s = pl.ds(c * _L, _L)
                dv = gs[b, e, s] - gd[b, e, s]
                for k in range(3):
                    vbuf[b, e, k, s] = vbuf[b, e, k, s] * 0.5 + uv[k] * dv
            return c2

        lax.fori_loop(0, _T, edge, 0)

    fire_inputs(0, 0)

    def pair(i2, carry):
        i = i2 * 2

        @pl.when(i2 > 0)
        def _():
            wait_out(i - 1, 1)
        fire_inputs(i + 1, 1)
        wait_inputs(i, 0)
        compute(i, 0)
        fire_out(i, 0)

        wait_out(i, 0)
        fire_inputs(i + 2, 0)
        wait_inputs(i + 1, 1)
        compute(i + 1, 1)
        fire_out(i + 1, 1)
        return carry

    lax.fori_loop(0, (_NT - 1) // 2, pair, 0)
    wait_out(_NT - 2, 1)
    wait_inputs(_NT - 1, 0)
    compute(_NT - 1, 0)
    fire_out(_NT - 1, 0)
    wait_out(_NT - 1, 0)


def kernel(v, x, edge_index, u, W, b):
    g = pl.pallas_call(
        _h_body,
        out_shape=jax.ShapeDtypeStruct((_N, _D), jnp.float32),
    )(x, W, b.reshape(1, _D))

    mesh = plsc.VectorSubcoreMesh(core_axis_name="c", subcore_axis_name="s")
    edge_fn = functools.partial(
        pl.kernel,
        mesh=mesh,
        out_type=jax.ShapeDtypeStruct((_E, 3, _D), jnp.float32),
        scratch_types=[
            pltpu.VMEM((_EW,), jnp.int32),
            pltpu.VMEM((_EW,), jnp.int32),
            pltpu.VMEM((_EW * 3 + _L,), jnp.float32),
            pltpu.VMEM((2, _T, _D), jnp.float32),
            pltpu.VMEM((2, _T, _D), jnp.float32),
            pltpu.VMEM((2, _T, 3, _D), jnp.float32),
            pltpu.SemaphoreType.DMA,
            pltpu.SemaphoreType.DMA,
            pltpu.SemaphoreType.DMA,
            pltpu.SemaphoreType.DMA,
            pltpu.SemaphoreType.DMA,
        ],
    )(_edge_body)
    return edge_fn(g, edge_index[0], edge_index[1], u.reshape(_E * 3), v)


# speedup vs baseline: 1.1827x; 1.0194x over previous
import functools

import jax
import jax.numpy as jnp
from jax import lax
from jax.experimental import pallas as pl
from jax.experimental.pallas import tpu as pltpu
from jax.experimental.pallas import tpu_sc as plsc

_N, _E, _D = 10000, 160000, 128
_NC = 2
_NS = 16
_NW = _NC * _NS
_EW = _E // _NW
_T = 40
_NT = _EW // _T
_L = 16
_EB = 1000


def _h_body(x_ref, w_ref, b_ref, o_ref):
    h = lax.dot_general(x_ref[...], w_ref[...], (((1,), (1,)), ((), ())),
                        preferred_element_type=jnp.float32)
    o_ref[...] = (h + b_ref[...]) * (1.0 / 512.0)


def _gather_body(g_hbm, src_hbm, dst_hbm, d_hbm,
                 sidx, didx, gs, gd, semin0, semin1, semo0, semo1):
    wid = lax.axis_index("s") * _NC + lax.axis_index("c")
    ebase = wid * _EW
    semin = (semin0, semin1)
    semo = (semo0, semo1)

    pltpu.sync_copy(src_hbm.at[pl.ds(ebase, _EW)], sidx)
    pltpu.sync_copy(dst_hbm.at[pl.ds(ebase, _EW)], didx)

    def fire_inputs(i, b):
        pltpu.async_copy(g_hbm.at[sidx.at[pl.ds(i * _T, _T)]], gs.at[b],
                         semin[b])
        pltpu.async_copy(g_hbm.at[didx.at[pl.ds(i * _T, _T)]], gd.at[b],
                         semin[b])

    def wait_inputs(i, b):
        pltpu.make_async_copy(g_hbm.at[sidx.at[pl.ds(i * _T, _T)]], gs.at[b],
                              semin[b]).wait()
        pltpu.make_async_copy(g_hbm.at[didx.at[pl.ds(i * _T, _T)]], gd.at[b],
                              semin[b]).wait()

    def fire_out(i, b):
        pltpu.async_copy(gs.at[b], d_hbm.at[pl.ds(ebase + i * _T, _T)],
                         semo[b])

    def wait_out(i, b):
        pltpu.make_async_copy(gs.at[b], d_hbm.at[pl.ds(ebase + i * _T, _T)],
                              semo[b]).wait()

    def compute(i, b):
        def edge(e, c2):
            for c in range(_D // _L):
                s = pl.ds(c * _L, _L)
                gs[b, e, s] = gs[b, e, s] - gd[b, e, s]
            return c2

        lax.fori_loop(0, _T, edge, 0)

    fire_inputs(0, 0)

    def pair(i2, carry):
        i = i2 * 2

        @pl.when(i2 > 0)
        def _():
            wait_out(i - 1, 1)
        fire_inputs(i + 1, 1)
        wait_inputs(i, 0)
        compute(i, 0)
        fire_out(i, 0)

        wait_out(i, 0)
        fire_inputs(i + 2, 0)
        wait_inputs(i + 1, 1)
        compute(i + 1, 1)
        fire_out(i + 1, 1)
        return carry

    lax.fori_loop(0, (_NT - 1) // 2, pair, 0)
    wait_out(_NT - 2, 1)
    wait_inputs(_NT - 1, 0)
    compute(_NT - 1, 0)
    fire_out(_NT - 1, 0)
    wait_out(_NT - 1, 0)


def _out_body(v_ref, d_ref, u0_ref, u1_ref, u2_ref, o_ref):
    d = d_ref[...]
    for k, uk in enumerate((u0_ref, u1_ref, u2_ref)):
        o_ref[:, k, :] = v_ref[:, k, :] * 0.5 + uk[...] * d


def kernel(v, x, edge_index, u, W, b):
    g = pl.pallas_call(
        _h_body,
        out_shape=jax.ShapeDtypeStruct((_N, _D), jnp.float32),
    )(x, W, b.reshape(1, _D))

    mesh = plsc.VectorSubcoreMesh(core_axis_name="c", subcore_axis_name="s")
    gather_fn = functools.partial(
        pl.kernel,
        mesh=mesh,
        out_type=jax.ShapeDtypeStruct((_E, _D), jnp.float32),
        scratch_types=[
            pltpu.VMEM((_EW,), jnp.int32),
            pltpu.VMEM((_EW,), jnp.int32),
            pltpu.VMEM((2, _T, _D), jnp.float32),
            pltpu.VMEM((2, _T, _D), jnp.float32),
            pltpu.SemaphoreType.DMA,
            pltpu.SemaphoreType.DMA,
            pltpu.SemaphoreType.DMA,
            pltpu.SemaphoreType.DMA,
        ],
    )(_gather_body)
    d = gather_fn(g, edge_index[0], edge_index[1])

    grid = _E // _EB
    out = pl.pallas_call(
        _out_body,
        grid=(grid,),
        in_specs=[
            pl.BlockSpec((_EB, 3, _D), lambda i: (i, 0, 0)),
            pl.BlockSpec((_EB, _D), lambda i: (i, 0)),
            pl.BlockSpec((_EB, 1), lambda i: (i, 0)),
            pl.BlockSpec((_EB, 1), lambda i: (i, 0)),
            pl.BlockSpec((_EB, 1), lambda i: (i, 0)),
        ],
        out_specs=pl.BlockSpec((_EB, 3, _D), lambda i: (i, 0, 0)),
        out_shape=jax.ShapeDtypeStruct((_E, 3, _D), jnp.float32),
    )(v, d, u[:, 0:1], u[:, 1:2], u[:, 2:3])
    return out


# speedup vs baseline: 3.1691x; 2.6795x over previous
import functools

import jax
import jax.numpy as jnp
from jax import lax
from jax.experimental import pallas as pl
from jax.experimental.pallas import tpu as pltpu
from jax.experimental.pallas import tpu_sc as plsc

_N, _E, _D = 10000, 160000, 128
_NC = 2
_NS = 16
_NW = _NC * _NS
_EW = _E // _NW
_T = 40
_NT = _EW // _T
_L = 16
_RB = 3000


def _h_body(x_ref, w_ref, b_ref, o_ref):
    h = lax.dot_general(x_ref[...], w_ref[...], (((1,), (1,)), ((), ())),
                        preferred_element_type=jnp.float32)
    o_ref[...] = (h + b_ref[...]) * (1.0 / 512.0)


def _gather_body(g_hbm, src_hbm, dst_hbm, u_hbm, d3_hbm,
                 sidx, didx, ub, gs, gd, dbuf,
                 semu, semin0, semin1, semo0, semo1):
    wid = lax.axis_index("s") * _NC + lax.axis_index("c")
    ebase = wid * _EW
    semin = (semin0, semin1)
    semo = (semo0, semo1)

    pltpu.sync_copy(src_hbm.at[pl.ds(ebase, _EW)], sidx)
    pltpu.sync_copy(dst_hbm.at[pl.ds(ebase, _EW)], didx)
    for k in range(3):
        pltpu.async_copy(u_hbm.at[pl.ds(k * _E + ebase, _EW)],
                         ub.at[pl.ds(k * (_EW + _L), _EW)], semu)
    for k in range(3):
        pltpu.make_async_copy(u_hbm.at[pl.ds(k * _E + ebase, _EW)],
                              ub.at[pl.ds(k * (_EW + _L), _EW)], semu).wait()

    def fire_inputs(i, b):
        pltpu.async_copy(g_hbm.at[sidx.at[pl.ds(i * _T, _T)]], gs.at[b],
                         semin[b])
        pltpu.async_copy(g_hbm.at[didx.at[pl.ds(i * _T, _T)]], gd.at[b],
                         semin[b])

    def wait_inputs(i, b):
        pltpu.make_async_copy(g_hbm.at[sidx.at[pl.ds(i * _T, _T)]], gs.at[b],
                              semin[b]).wait()
        pltpu.make_async_copy(g_hbm.at[didx.at[pl.ds(i * _T, _T)]], gd.at[b],
                              semin[b]).wait()

    def fire_out(i, b):
        for k in range(3):
            pltpu.async_copy(dbuf.at[b, k],
                             d3_hbm.at[pl.ds(k * _E + ebase + i * _T, _T)],
                             semo[b])

    def wait_out(i, b):
        for k in range(3):
            pltpu.make_async_copy(
                dbuf.at[b, k],
                d3_hbm.at[pl.ds(k * _E + ebase + i * _T, _T)],
                semo[b]).wait()

    def compute(i, b):
        def edge(e, c2):
            uv = [ub[pl.ds(k * (_EW + _L) + i * _T + e, _L)][0]
                  for k in range(3)]
            for c in range(_D // _L):
                s = pl.ds(c * _L, _L)
                dv = gs[b, e, s] - gd[b, e, s]
                for k in range(3):
                    dbuf[b, k, e, s] = uv[k] * dv
            return c2

        lax.fori_loop(0, _T, edge, 0)

    fire_inputs(0, 0)

    def pair(i2, carry):
        i = i2 * 2

        @pl.when(i2 > 0)
        def _():
            wait_out(i - 1, 1)
        fire_inputs(i + 1, 1)
        wait_inputs(i, 0)
        compute(i, 0)
        fire_out(i, 0)

        wait_out(i, 0)
        fire_inputs(i + 2, 0)
        wait_inputs(i + 1, 1)
        compute(i + 1, 1)
        fire_out(i + 1, 1)
        return carry

    lax.fori_loop(0, (_NT - 1) // 2, pair, 0)
    wait_out(_NT - 2, 1)
    wait_inputs(_NT - 1, 0)
    compute(_NT - 1, 0)
    fire_out(_NT - 1, 0)
    wait_out(_NT - 1, 0)


def _res_body(v_ref, d_ref, o_ref):
    o_ref[...] = v_ref[...] * 0.5 + d_ref[...]


def kernel(v, x, edge_index, u, W, b):
    g = pl.pallas_call(
        _h_body,
        out_shape=jax.ShapeDtypeStruct((_N, _D), jnp.float32),
    )(x, W, b.reshape(1, _D))

    uflat = u.T.reshape(3 * _E)

    mesh = plsc.VectorSubcoreMesh(core_axis_name="c", subcore_axis_name="s")
    gather_fn = functools.partial(
        pl.kernel,
        mesh=mesh,
        out_type=jax.ShapeDtypeStruct((3 * _E, _D), jnp.float32),
        scratch_types=[
            pltpu.VMEM((_EW,), jnp.int32),
            pltpu.VMEM((_EW,), jnp.int32),
            pltpu.VMEM((3 * (_EW + _L),), jnp.float32),
            pltpu.VMEM((2, _T, _D), jnp.float32),
            pltpu.VMEM((2, _T, _D), jnp.float32),
            pltpu.VMEM((2, 3, _T, _D), jnp.float32),
            pltpu.SemaphoreType.DMA,
            pltpu.SemaphoreType.DMA,
            pltpu.SemaphoreType.DMA,
            pltpu.SemaphoreType.DMA,
            pltpu.SemaphoreType.DMA,
        ],
    )(_gather_body)
    d3 = gather_fn(g, edge_index[0], edge_index[1], uflat)

    v2 = v.transpose(1, 0, 2).reshape(3 * _E, _D)
    grid = (3 * _E) // _RB
    out2 = pl.pallas_call(
        _res_body,
        grid=(grid,),
        in_specs=[
            pl.BlockSpec((_RB, _D), lambda i: (i, 0)),
            pl.BlockSpec((_RB, _D), lambda i: (i, 0)),
        ],
        out_specs=pl.BlockSpec((_RB, _D), lambda i: (i, 0)),
        out_shape=jax.ShapeDtypeStruct((3 * _E, _D), jnp.float32),
    )(v2, d3)
    return out2.reshape(3, _E, _D).transpose(1, 0, 2)


# speedup vs baseline: 5.1557x; 1.6269x over previous
import functools

import jax
import jax.numpy as jnp
from jax import lax
from jax.experimental import pallas as pl
from jax.experimental.pallas import tpu as pltpu
from jax.experimental.pallas import tpu_sc as plsc

_N, _E, _D = 10000, 160000, 128
_NC = 2
_NS = 16
_NW = _NC * _NS
_EW = _E // _NW
_T = 40
_NT = _EW // _T
_L = 16


def _h_body(x_ref, w_ref, b_ref, o_ref):
    h = lax.dot_general(x_ref[...], w_ref[...], (((1,), (1,)), ((), ())),
                        preferred_element_type=jnp.float32)
    o_ref[...] = (h + b_ref[...]) * (1.0 / 512.0)


def _edge_body(g_hbm, src_hbm, dst_hbm, u_hbm, v_hbm, o_hbm,
               sidx, didx, ub, gs, gd, vbuf,
               semu, semin0, semin1, semo0, semo1):
    wid = lax.axis_index("s") * _NC + lax.axis_index("c")
    ebase = wid * _EW
    semin = (semin0, semin1)
    semo = (semo0, semo1)

    pltpu.sync_copy(src_hbm.at[pl.ds(ebase, _EW)], sidx)
    pltpu.sync_copy(dst_hbm.at[pl.ds(ebase, _EW)], didx)
    for k in range(3):
        pltpu.async_copy(u_hbm.at[pl.ds(k * _E + ebase, _EW)],
                         ub.at[pl.ds(k * (_EW + _L), _EW)], semu)
    for k in range(3):
        pltpu.make_async_copy(u_hbm.at[pl.ds(k * _E + ebase, _EW)],
                              ub.at[pl.ds(k * (_EW + _L), _EW)], semu).wait()

    def fire_inputs(i, b):
        pltpu.async_copy(g_hbm.at[sidx.at[pl.ds(i * _T, _T)]], gs.at[b],
                         semin[b])
        pltpu.async_copy(g_hbm.at[didx.at[pl.ds(i * _T, _T)]], gd.at[b],
                         semin[b])
        for k in range(3):
            pltpu.async_copy(v_hbm.at[pl.ds(k * _E + ebase + i * _T, _T)],
                             vbuf.at[b, k], semin[b])

    def wait_inputs(i, b):
        pltpu.make_async_copy(g_hbm.at[sidx.at[pl.ds(i * _T, _T)]], gs.at[b],
                              semin[b]).wait()
        pltpu.make_async_copy(g_hbm.at[didx.at[pl.ds(i * _T, _T)]], gd.at[b],
                              semin[b]).wait()
        for k in range(3):
            pltpu.make_async_copy(
                v_hbm.at[pl.ds(k * _E + ebase + i * _T, _T)],
                vbuf.at[b, k], semin[b]).wait()

    def fire_out(i, b):
        for k in range(3):
            pltpu.async_copy(vbuf.at[b, k],
                             o_hbm.at[pl.ds(k * _E + ebase + i * _T, _T)],
                             semo[b])

    def wait_out(i, b):
        for k in range(3):
            pltpu.make_async_copy(
                vbuf.at[b, k],
                o_hbm.at[pl.ds(k * _E + ebase + i * _T, _T)],
                semo[b]).wait()

    def compute(i, b):
        def edge(e, c2):
            uv = [ub[pl.ds(k * (_EW + _L) + i * _T + e, _L)][0]
                  for k in range(3)]
            for c in range(_D // _L):
                s = pl.ds(c * _L, _L)
                dv = gs[b, e, s] - gd[b, e, s]
                for k in range(3):
                    vbuf[b, k, e, s] = vbuf[b, k, e, s] * 0.5 + uv[k] * dv
            return c2

        lax.fori_loop(0, _T, edge, 0)

    fire_inputs(0, 0)

    def pair(i2, carry):
        i = i2 * 2

        @pl.when(i2 > 0)
        def _():
            wait_out(i - 1, 1)
        fire_inputs(i + 1, 1)
        wait_inputs(i, 0)
        compute(i, 0)
        fire_out(i, 0)

        wait_out(i, 0)
        fire_inputs(i + 2, 0)
        wait_inputs(i + 1, 1)
        compute(i + 1, 1)
        fire_out(i + 1, 1)
        return carry

    lax.fori_loop(0, (_NT - 1) // 2, pair, 0)
    wait_out(_NT - 2, 1)
    wait_inputs(_NT - 1, 0)
    compute(_NT - 1, 0)
    fire_out(_NT - 1, 0)
    wait_out(_NT - 1, 0)


def kernel(v, x, edge_index, u, W, b):
    g = pl.pallas_call(
        _h_body,
        out_shape=jax.ShapeDtypeStruct((_N, _D), jnp.float32),
    )(x, W, b.reshape(1, _D))

    uflat = u.T.reshape(3 * _E)
    v2 = v.transpose(1, 0, 2).reshape(3 * _E, _D)

    mesh = plsc.VectorSubcoreMesh(core_axis_name="c", subcore_axis_name="s")
    edge_fn = functools.partial(
        pl.kernel,
        mesh=mesh,
        out_type=jax.ShapeDtypeStruct((3 * _E, _D), jnp.float32),
        scratch_types=[
            pltpu.VMEM((_EW,), jnp.int32),
            pltpu.VMEM((_EW,), jnp.int32),
            pltpu.VMEM((3 * (_EW + _L),), jnp.float32),
            pltpu.VMEM((2, _T, _D), jnp.float32),
            pltpu.VMEM((2, _T, _D), jnp.float32),
            pltpu.VMEM((2, 3, _T, _D), jnp.float32),
            pltpu.SemaphoreType.DMA,
            pltpu.SemaphoreType.DMA,
            pltpu.SemaphoreType.DMA,
            pltpu.SemaphoreType.DMA,
            pltpu.SemaphoreType.DMA,
        ],
    )(_edge_body)
    out2 = edge_fn(g, edge_index[0], edge_index[1], uflat, v2)
    return out2.reshape(3, _E, _D).transpose(1, 0, 2)
